# baseline (speedup 1.0000x reference)
import functools

import jax
import jax.numpy as jnp
from jax import lax
from jax.experimental import pallas as pl
from jax.experimental.pallas import tpu as pltpu
from jax.experimental.pallas import tpu_sc as plsc

B = 4096
L = 200
D = 64
N_TOTAL = B * L
NC, NS = 2, 16
NW = NC * NS
PER_W = N_TOTAL // NW
G = 128
SET = 5
N_SUPER = PER_W // (G * SET)
PAIR = 2 * G

_mesh = plsc.VectorSubcoreMesh(core_axis_name="c", subcore_axis_name="s")


@functools.partial(
    pl.kernel,
    out_type=jax.ShapeDtypeStruct((N_TOTAL, D), jnp.float32),
    mesh=_mesh,
    scratch_types=[
        pltpu.VMEM((2, SET * PAIR), jnp.int32),
        pltpu.VMEM((2, SET * G, D), jnp.float32),
        pltpu.VMEM((3 * D,), jnp.float32),
        pltpu.SemaphoreType.DMA,
        pltpu.SemaphoreType.DMA,
        pltpu.SemaphoreType.DMA,
        pltpu.SemaphoreType.DMA,
        pltpu.SemaphoreType.DMA,
    ],
    compiler_params=pltpu.CompilerParams(use_tc_tiling_on_sc=False,
                                         needs_layout_passes=False),
)
def _embed(pairs_hbm, tok_hbm, rt_hbm, out_hbm, idx_v, rows_v, rt_v,
           sem_i0, sem_i1, sem_tok, sem_w0, sem_w1):
    wid = lax.axis_index("s") * NC + lax.axis_index("c")
    base_w = wid * PER_W
    pair_w = base_w * 2
    pltpu.sync_copy(rt_hbm, rt_v)

    sem_i = (sem_i0, sem_i1)
    sem_w = (sem_w0, sem_w1)

    def idx_copy(s, p):
        pltpu.async_copy(
            pairs_hbm.at[pl.ds(pair_w + s * SET * PAIR, SET * PAIR)],
            idx_v.at[p], sem_i[p])

    def idx_wait(p):
        pltpu.make_async_copy(
            pairs_hbm.at[pl.ds(0, SET * PAIR)], idx_v.at[p], sem_i[p]).wait()

    def fill_roles(p):
        iota16 = lax.iota(jnp.int32, 16)
        zero16 = iota16 * 0

        @pl.loop(0, SET * (G // 16))
        def _blk(t):
            off = (t // 8) * PAIR + G + (t % 8) * 16
            roles = idx_v[p, pl.ds(off, 16)]
            rbase = roles * D
            rows16 = t * 16 + iota16
            for c in range(D):
                vals = plsc.load_gather(rt_v, [rbase + c])
                plsc.store_scatter(rows_v.at[p], [rows16, zero16 + c], vals)

    def drain_wb(p):
        for b in range(SET):
            pltpu.make_async_copy(rows_v.at[p, pl.ds(b * G, G)],
                                  out_hbm.at[pl.ds(0, G)], sem_w[p]).wait()

    idx_copy(0, 0)
    idx_wait(0)
    fill_roles(0)
    idx_copy(1, 1)

    @pl.loop(0, N_SUPER, step=2)
    def _super(s0):
        for p in range(2):
            s = s0 + p
            q = 1 - p
            base = base_w + s * (SET * G)
            cps = [pltpu.async_copy(
                       tok_hbm.at[idx_v.at[p, pl.ds(b * PAIR, G)]],
                       rows_v.at[p, pl.ds(b * G, G)], sem_tok, add=True)
                   for b in range(SET)]

            @pl.when(s >= 1)
            def _():
                drain_wb(q)

            @pl.when(s + 1 < N_SUPER)
            def _():
                idx_wait(q)
                fill_roles(q)

            for cp in cps:
                cp.wait()
            @pl.when(s + 2 < N_SUPER)
            def _():
                idx_copy(s + 2, p)
            for b in range(SET):
                pltpu.async_copy(rows_v.at[p, pl.ds(b * G, G)],
                                 out_hbm.at[pl.ds(base + b * G, G)], sem_w[p])

    drain_wb((N_SUPER - 1) % 2)


def kernel(idx, role_ids, token_table, role_table):
    idx2d = idx.reshape(N_TOTAL // G, G).astype(jnp.int32)
    role2d = role_ids.reshape(N_TOTAL // G, G).astype(jnp.int32)
    pairs = jnp.stack([idx2d, role2d], axis=1).reshape(-1)
    out = _embed(pairs, token_table, role_table.reshape(-1))
    return out.reshape(B, L, D)


# speedup vs baseline: 2.3808x; 2.3808x over previous
import functools

import jax
import jax.numpy as jnp
from jax import lax
from jax.experimental import pallas as pl
from jax.experimental.pallas import tpu as pltpu
from jax.experimental.pallas import tpu_sc as plsc

B = 4096
L = 200
D = 64
N_TOTAL = B * L
NC, NS = 2, 16
NW = NC * NS
PER_W = N_TOTAL // NW
G = 128
SET = 5
N_SUPER = PER_W // (G * SET)

_mesh = plsc.VectorSubcoreMesh(core_axis_name="c", subcore_axis_name="s")


@functools.partial(
    pl.kernel,
    out_type=jax.ShapeDtypeStruct((N_TOTAL, D), jnp.float32),
    mesh=_mesh,
    scratch_types=[
        pltpu.VMEM((2, SET, 2, G), jnp.int32),
        pltpu.VMEM((2, SET, G, D), jnp.float32),
        pltpu.VMEM_SHARED((3, D), jnp.float32),
        pltpu.SemaphoreType.DMA,
        pltpu.SemaphoreType.DMA,
        pltpu.SemaphoreType.DMA,
        pltpu.SemaphoreType.DMA,
        pltpu.SemaphoreType.DMA,
        pltpu.SemaphoreType.DMA,
    ],
    compiler_params=pltpu.CompilerParams(use_tc_tiling_on_sc=False),
)
def _embed(pairs_hbm, tok_hbm, rt_hbm, out_hbm, idx_v, rows_v, rt_sh,
           sem_i0, sem_i1, sem_role, sem_tok, sem_w0, sem_w1):
    wid = lax.axis_index("s") * NC + lax.axis_index("c")
    base_w = wid * PER_W
    grp_w = base_w // G
    pltpu.sync_copy(rt_hbm, rt_sh)

    sem_i = (sem_i0, sem_i1)
    sem_w = (sem_w0, sem_w1)

    def idx_copy(s, p):
        pltpu.async_copy(
            pairs_hbm.at[pl.ds(grp_w + s * SET, SET)], idx_v.at[p], sem_i[p])

    def idx_wait(p):
        pltpu.make_async_copy(
            pairs_hbm.at[pl.ds(0, SET)], idx_v.at[p], sem_i[p]).wait()

    def fire_roles(p):
        for b in range(SET):
            pltpu.async_copy(rt_sh.at[idx_v.at[p, b, 1]], rows_v.at[p, b],
                             sem_role)

    def drain_roles(p):
        for b in range(SET):
            pltpu.make_async_copy(rt_sh.at[idx_v.at[p, b, 1]],
                                  rows_v.at[p, b], sem_role).wait()

    def drain_wb(p):
        for b in range(SET):
            pltpu.make_async_copy(rows_v.at[p, b], out_hbm.at[pl.ds(0, G)],
                                  sem_w[p]).wait()

    idx_copy(0, 0)
    idx_wait(0)
    fire_roles(0)
    idx_copy(1, 1)

    @pl.loop(0, N_SUPER, step=2)
    def _super(s0):
        for p in range(2):
            s = s0 + p
            q = 1 - p
            base = base_w + s * (SET * G)
            drain_roles(p)
            cps = [pltpu.async_copy(tok_hbm.at[idx_v.at[p, b, 0]],
                                    rows_v.at[p, b], sem_tok, add=True)
                   for b in range(SET)]

            @pl.when(s >= 1)
            def _():
                drain_wb(q)

            @pl.when(s + 1 < N_SUPER)
            def _():
                idx_wait(q)
                fire_roles(q)

            for cp in cps:
                cp.wait()
            @pl.when(s + 2 < N_SUPER)
            def _():
                idx_copy(s + 2, p)
            for b in range(SET):
                pltpu.async_copy(rows_v.at[p, b],
                                 out_hbm.at[pl.ds(base + b * G, G)], sem_w[p])

    drain_wb((N_SUPER - 1) % 2)


def kernel(idx, role_ids, token_table, role_table):
    idx2d = idx.reshape(N_TOTAL // G, G).astype(jnp.int32)
    role2d = role_ids.reshape(N_TOTAL // G, G).astype(jnp.int32)
    pairs = jnp.stack([idx2d, role2d], axis=1)
    out = _embed(pairs, token_table, role_table)
    return out.reshape(B, L, D)
